# Initial kernel scaffold; baseline (speedup 1.0000x reference)
import functools

import jax
import jax.numpy as jnp
from jax import lax
from jax.experimental import pallas as pl
from jax.experimental.pallas import tpu as pltpu
from jax.experimental.pallas import tpu_sc as plsc

NC = 2
NS = 16
NW = NC * NS
CHUNK = 128
TRASH = 512


def _seg_sum(N, Npad, K, D):
    NZT = Npad // NS
    NOT_ = N // NS
    mesh = plsc.VectorSubcoreMesh(core_axis_name="c", subcore_axis_name="s")

    @functools.partial(
        pl.kernel,
        out_type=jax.ShapeDtypeStruct((NC, N, D), jnp.float32),
        mesh=mesh,
        scratch_types=[
            pltpu.VMEM((K, CHUNK), jnp.int32),
            pltpu.VMEM((K, CHUNK), jnp.int32),
            pltpu.VMEM((CHUNK, D), jnp.float32),
            pltpu.VMEM((NZT, D), jnp.float32),
            pltpu.VMEM_SHARED((Npad, D), jnp.float32),
            pltpu.SemaphoreType.DMA,
        ],
    )
    def kfn(g_hbm, src_hbm, dst_hbm, zeros_hbm, out_hbm,
            src_v, dst_v, buf, zslab, acc, sem):
        cid = lax.axis_index("c")
        sid = lax.axis_index("s")
        wid = cid * NS + sid
        pltpu.sync_copy(src_hbm.at[wid], src_v)
        pltpu.sync_copy(dst_hbm.at[wid], dst_v)
        pltpu.sync_copy(zeros_hbm, zslab)
        pltpu.sync_copy(zslab, acc.at[pl.ds(sid * NZT, NZT)])
        plsc.subcore_barrier()

        def body(k, carry):
            pltpu.async_copy(g_hbm.at[src_v.at[k]], buf, sem).wait()
            pltpu.sync_copy(buf, acc.at[dst_v.at[k]], add=True)
            return carry

        lax.fori_loop(0, K, body, 0)
        plsc.subcore_barrier()
        pltpu.sync_copy(acc.at[pl.ds(sid * NOT_, NOT_)], zslab.at[pl.ds(0, NOT_)])
        pltpu.sync_copy(zslab.at[pl.ds(0, NOT_)], out_hbm.at[cid].at[pl.ds(sid * NOT_, NOT_)])

    return kfn


def _deg_count(N, Npad, K):
    D = 16
    NZT = Npad // NS
    NOT_ = N // NS
    mesh = plsc.VectorSubcoreMesh(core_axis_name="c", subcore_axis_name="s")

    @functools.partial(
        pl.kernel,
        out_type=jax.ShapeDtypeStruct((NC, N, D), jnp.float32),
        mesh=mesh,
        scratch_types=[
            pltpu.VMEM((K, CHUNK), jnp.int32),
            pltpu.VMEM((CHUNK, D), jnp.float32),
            pltpu.VMEM((NZT, D), jnp.float32),
            pltpu.VMEM_SHARED((Npad, D), jnp.float32),
        ],
    )
    def kfn(dst_hbm, ones_hbm, zeros_hbm, out_hbm, dst_v, ones_v, zslab, acc):
        cid = lax.axis_index("c")
        sid = lax.axis_index("s")
        wid = cid * NS + sid
        pltpu.sync_copy(dst_hbm.at[wid], dst_v)
        pltpu.sync_copy(ones_hbm, ones_v)
        pltpu.sync_copy(zeros_hbm, zslab)
        pltpu.sync_copy(zslab, acc.at[pl.ds(sid * NZT, NZT)])
        plsc.subcore_barrier()

        def body(k, carry):
            pltpu.sync_copy(ones_v, acc.at[dst_v.at[k]], add=True)
            return carry

        lax.fori_loop(0, K, body, 0)
        plsc.subcore_barrier()
        pltpu.sync_copy(acc.at[pl.ds(sid * NOT_, NOT_)], zslab.at[pl.ds(0, NOT_)])
        pltpu.sync_copy(zslab.at[pl.ds(0, NOT_)], out_hbm.at[cid].at[pl.ds(sid * NOT_, NOT_)])

    return kfn


def _tc_scale_mm(x, W, d0, d1):
    N, D_out = x.shape[0], W.shape[1]

    def body(x_ref, w_ref, d0_ref, d1_ref, g_ref, dinv_ref):
        deg = d0_ref[...] + d1_ref[...]
        dinv = jnp.where(deg > 0, lax.rsqrt(deg), 0.0)
        h = jnp.dot(x_ref[...], w_ref[...], preferred_element_type=jnp.float32)
        g_ref[...] = h * dinv
        dinv_ref[...] = dinv

    return pl.pallas_call(
        body,
        out_shape=(jax.ShapeDtypeStruct((N, D_out), jnp.float32),
                   jax.ShapeDtypeStruct((N, 1), jnp.float32)),
    )(x, W, d0, d1)


def _tc_mid(p0, p1, dinv, b1, W2):
    N, D_out = p0.shape[0], W2.shape[1]

    def body(p0_ref, p1_ref, dinv_ref, b1_ref, w2_ref, g_ref):
        dinv = dinv_ref[...]
        h = jnp.maximum(dinv * (p0_ref[...] + p1_ref[...]) + b1_ref[...], 0.0)
        g_ref[...] = dinv * jnp.dot(h, w2_ref[...], preferred_element_type=jnp.float32)

    return pl.pallas_call(
        body,
        out_shape=jax.ShapeDtypeStruct((N, D_out), jnp.float32),
    )(p0, p1, dinv, b1, W2)


def _tc_final(p0, p1, dinv, b2):
    N, D_out = p0.shape

    def body(p0_ref, p1_ref, dinv_ref, b2_ref, o_ref):
        o_ref[...] = dinv_ref[...] * (p0_ref[...] + p1_ref[...]) + b2_ref[...]

    return pl.pallas_call(
        body,
        out_shape=jax.ShapeDtypeStruct((N, D_out), jnp.float32),
    )(p0, p1, dinv, b2)


def kernel(x, edge_index, W1, b1, W2, b2):
    N = x.shape[0]
    E = edge_index.shape[1]
    D_HID = W1.shape[1]
    D_OUT = W2.shape[1]
    Npad = N + TRASH

    per_w = -(-E // NW)
    K = -(-per_w // CHUNK)
    Ep = NW * K * CHUNK
    pad = Ep - E
    it = jnp.arange(pad, dtype=jnp.int32)
    src_p = jnp.concatenate([edge_index[0], it % N]).reshape(NW, K, CHUNK)
    dst_p = jnp.concatenate([edge_index[1], N + (it % TRASH)]).reshape(NW, K, CHUNK)

    NZT = Npad // NS
    ones16 = jnp.ones((CHUNK, 16), jnp.float32)
    zeros16 = jnp.zeros((NZT, 16), jnp.float32)
    zeros_h = jnp.zeros((NZT, D_HID), jnp.float32)
    zeros_o = jnp.zeros((NZT, D_OUT), jnp.float32)

    pdeg = _deg_count(N, Npad, K)(dst_p, ones16, zeros16)
    g1, dinv = _tc_scale_mm(x, W1, pdeg[0, :, 0:1], pdeg[1, :, 0:1])
    p1 = _seg_sum(N, Npad, K, D_HID)(g1, src_p, dst_p, zeros_h)
    g2 = _tc_mid(p1[0], p1[1], dinv, b1.reshape(1, -1), W2)
    p2 = _seg_sum(N, Npad, K, D_OUT)(g2, src_p, dst_p, zeros_o)
    out = _tc_final(p2[0], p2[1], dinv, b2.reshape(1, -1))
    return (out, 0)


# speedup vs baseline: 26.5232x; 26.5232x over previous
import functools

import jax
import jax.numpy as jnp
from jax import lax
from jax.experimental import pallas as pl
from jax.experimental.pallas import tpu as pltpu
from jax.experimental.pallas import tpu_sc as plsc

NC = 2
NS = 16
NW = NC * NS
CHUNK = 128


def _seg_sum(Npad, K, D):
    NZT = Npad // NS
    mesh = plsc.VectorSubcoreMesh(core_axis_name="c", subcore_axis_name="s")

    @functools.partial(
        pl.kernel,
        out_type=jax.ShapeDtypeStruct((NC, Npad, D), jnp.float32),
        mesh=mesh,
        compiler_params=pltpu.CompilerParams(use_tc_tiling_on_sc=False),
        scratch_types=[
            pltpu.VMEM((K, CHUNK), jnp.int32),
            pltpu.VMEM((K, CHUNK), jnp.int32),
            pltpu.VMEM((CHUNK, D), jnp.float32),
            pltpu.VMEM((NZT, D), jnp.float32),
            pltpu.VMEM_SHARED((Npad, D), jnp.float32),
            pltpu.SemaphoreType.DMA,
        ],
    )
    def kfn(g_hbm, src_hbm, dst_hbm, zeros_hbm, out_hbm,
            src_v, dst_v, buf, zslab, acc, sem):
        cid = lax.axis_index("c")
        sid = lax.axis_index("s")
        wid = cid * NS + sid
        pltpu.sync_copy(src_hbm.at[wid], src_v)
        pltpu.sync_copy(dst_hbm.at[wid], dst_v)
        pltpu.sync_copy(zeros_hbm, zslab)
        pltpu.sync_copy(zslab, acc.at[pl.ds(sid * NZT, NZT)])
        plsc.subcore_barrier()

        def body(k, carry):
            pltpu.async_copy(g_hbm.at[src_v.at[k]], buf, sem).wait()
            pltpu.sync_copy(buf, acc.at[dst_v.at[k]], add=True)
            return carry

        lax.fori_loop(0, K, body, 0)
        plsc.subcore_barrier()
        pltpu.sync_copy(acc.at[pl.ds(sid * NZT, NZT)], zslab)
        pltpu.sync_copy(zslab, out_hbm.at[cid].at[pl.ds(sid * NZT, NZT)])

    return kfn


def _deg_count(Npad, K):
    D = 16
    NZT = Npad // NS
    mesh = plsc.VectorSubcoreMesh(core_axis_name="c", subcore_axis_name="s")

    @functools.partial(
        pl.kernel,
        out_type=jax.ShapeDtypeStruct((NC, Npad, D), jnp.float32),
        mesh=mesh,
        compiler_params=pltpu.CompilerParams(use_tc_tiling_on_sc=False),
        scratch_types=[
            pltpu.VMEM((K, CHUNK), jnp.int32),
            pltpu.VMEM((CHUNK, D), jnp.float32),
            pltpu.VMEM((NZT, D), jnp.float32),
            pltpu.VMEM_SHARED((Npad, D), jnp.float32),
        ],
    )
    def kfn(dst_hbm, ones_hbm, zeros_hbm, out_hbm, dst_v, ones_v, zslab, acc):
        cid = lax.axis_index("c")
        sid = lax.axis_index("s")
        wid = cid * NS + sid
        pltpu.sync_copy(dst_hbm.at[wid], dst_v)
        pltpu.sync_copy(ones_hbm, ones_v)
        pltpu.sync_copy(zeros_hbm, zslab)
        pltpu.sync_copy(zslab, acc.at[pl.ds(sid * NZT, NZT)])
        plsc.subcore_barrier()

        def body(k, carry):
            pltpu.sync_copy(ones_v, acc.at[dst_v.at[k]], add=True)
            return carry

        lax.fori_loop(0, K, body, 0)
        plsc.subcore_barrier()
        pltpu.sync_copy(acc.at[pl.ds(sid * NZT, NZT)], zslab)
        pltpu.sync_copy(zslab, out_hbm.at[cid].at[pl.ds(sid * NZT, NZT)])

    return kfn


def _tc_scale_mm(x, W, d0, d1):
    N, D_out = x.shape[0], W.shape[1]

    def body(x_ref, w_ref, d0_ref, d1_ref, g_ref, dinv_ref):
        deg = d0_ref[...] + d1_ref[...]
        dinv = jnp.where(deg > 0, lax.rsqrt(deg), 0.0)
        h = jnp.dot(x_ref[...], w_ref[...], preferred_element_type=jnp.float32)
        g_ref[...] = h * dinv
        dinv_ref[...] = dinv

    return pl.pallas_call(
        body,
        out_shape=(jax.ShapeDtypeStruct((N, D_out), jnp.float32),
                   jax.ShapeDtypeStruct((N, 1), jnp.float32)),
    )(x, W, d0, d1)


def _tc_mid(p0, p1, dinv, b1, W2):
    N, D_out = p0.shape[0], W2.shape[1]

    def body(p0_ref, p1_ref, dinv_ref, b1_ref, w2_ref, g_ref):
        dinv = dinv_ref[...]
        h = jnp.maximum(dinv * (p0_ref[...] + p1_ref[...]) + b1_ref[...], 0.0)
        g_ref[...] = dinv * jnp.dot(h, w2_ref[...], preferred_element_type=jnp.float32)

    return pl.pallas_call(
        body,
        out_shape=jax.ShapeDtypeStruct((N, D_out), jnp.float32),
    )(p0, p1, dinv, b1, W2)


def _tc_final(p0, p1, dinv, b2):
    N, D_out = p0.shape

    def body(p0_ref, p1_ref, dinv_ref, b2_ref, o_ref):
        o_ref[...] = dinv_ref[...] * (p0_ref[...] + p1_ref[...]) + b2_ref[...]

    return pl.pallas_call(
        body,
        out_shape=jax.ShapeDtypeStruct((N, D_out), jnp.float32),
    )(p0, p1, dinv, b2)


def kernel(x, edge_index, W1, b1, W2, b2):
    N = x.shape[0]
    E = edge_index.shape[1]
    D_HID = W1.shape[1]
    D_OUT = W2.shape[1]
    Npad = (N // 128 + 1) * 128
    trash = Npad - N

    per_w = -(-E // NW)
    K = -(-per_w // CHUNK)
    Ep = NW * K * CHUNK
    pad = Ep - E
    it = jnp.arange(pad, dtype=jnp.int32)
    src_p = jnp.concatenate([edge_index[0], it % N]).reshape(NW, K, CHUNK)
    dst_p = jnp.concatenate([edge_index[1], N + (it % trash)]).reshape(NW, K, CHUNK)

    NZT = Npad // NS
    ones16 = jnp.ones((CHUNK, 16), jnp.float32)
    zeros16 = jnp.zeros((NZT, 16), jnp.float32)
    zeros_h = jnp.zeros((NZT, D_HID), jnp.float32)
    zeros_o = jnp.zeros((NZT, D_OUT), jnp.float32)

    pdeg = _deg_count(Npad, K)(dst_p, ones16, zeros16)
    g1, dinv = _tc_scale_mm(x, W1, pdeg[0, :N, 0:1], pdeg[1, :N, 0:1])
    p1 = _seg_sum(Npad, K, D_HID)(g1, src_p, dst_p, zeros_h)
    g2 = _tc_mid(p1[0, :N], p1[1, :N], dinv, b1.reshape(1, -1), W2)
    p2 = _seg_sum(Npad, K, D_OUT)(g2, src_p, dst_p, zeros_o)
    out = _tc_final(p2[0, :N], p2[1, :N], dinv, b2.reshape(1, -1))
    return (out, 0)


# speedup vs baseline: 34.4910x; 1.3004x over previous
import functools

import jax
import jax.numpy as jnp
from jax import lax
from jax.experimental import pallas as pl
from jax.experimental.pallas import tpu as pltpu
from jax.experimental.pallas import tpu_sc as plsc

NC = 2
NS = 16
NW = NC * NS
CHUNK = 128


def _seg_sum(Npad, K, D):
    NZT = Npad // NS
    mesh = plsc.VectorSubcoreMesh(core_axis_name="c", subcore_axis_name="s")

    @functools.partial(
        pl.kernel,
        out_type=jax.ShapeDtypeStruct((NC, Npad, D), jnp.float32),
        mesh=mesh,
        compiler_params=pltpu.CompilerParams(use_tc_tiling_on_sc=False),
        scratch_types=[
            pltpu.VMEM((K, CHUNK), jnp.int32),
            pltpu.VMEM((K, CHUNK), jnp.int32),
            pltpu.VMEM((CHUNK, D), jnp.float32),
            pltpu.VMEM((CHUNK, D), jnp.float32),
            pltpu.VMEM((NZT, D), jnp.float32),
            pltpu.VMEM_SHARED((Npad, D), jnp.float32),
            pltpu.SemaphoreType.DMA,
            pltpu.SemaphoreType.DMA,
            pltpu.SemaphoreType.DMA,
            pltpu.SemaphoreType.DMA,
        ],
    )
    def kfn(g_hbm, src_hbm, dst_hbm, zeros_hbm, out_hbm,
            src_v, dst_v, buf0, buf1, zslab, acc, sem0, sem1, semA, semB):
        cid = lax.axis_index("c")
        sid = lax.axis_index("s")
        wid = cid * NS + sid
        pltpu.async_copy(src_hbm.at[wid], src_v, semA)
        pltpu.async_copy(dst_hbm.at[wid], dst_v, semB)
        pltpu.sync_copy(zeros_hbm, zslab)
        pltpu.sync_copy(zslab, acc.at[pl.ds(sid * NZT, NZT)])
        pltpu.make_async_copy(src_hbm.at[wid], src_v, semA).wait()
        pltpu.make_async_copy(dst_hbm.at[wid], dst_v, semB).wait()
        plsc.subcore_barrier()

        P = K // 2
        pltpu.async_copy(g_hbm.at[src_v.at[0]], buf0, sem0)

        def body(p, carry):
            k0 = 2 * p
            pltpu.async_copy(g_hbm.at[src_v.at[k0 + 1]], buf1, sem1)
            pltpu.make_async_copy(g_hbm.at[src_v.at[0]], buf0, sem0).wait()
            pltpu.sync_copy(buf0, acc.at[dst_v.at[k0]], add=True)
            pltpu.async_copy(g_hbm.at[src_v.at[jnp.minimum(k0 + 2, K - 1)]],
                             buf0, sem0)
            pltpu.make_async_copy(g_hbm.at[src_v.at[0]], buf1, sem1).wait()
            pltpu.sync_copy(buf1, acc.at[dst_v.at[k0 + 1]], add=True)
            return carry

        lax.fori_loop(0, P, body, 0)
        pltpu.make_async_copy(g_hbm.at[src_v.at[0]], buf0, sem0).wait()
        plsc.subcore_barrier()
        pltpu.sync_copy(acc.at[pl.ds(sid * NZT, NZT)], zslab)
        pltpu.sync_copy(zslab, out_hbm.at[cid].at[pl.ds(sid * NZT, NZT)])

    return kfn


def _deg_count(Npad, K):
    D = 16
    NZT = Npad // NS
    mesh = plsc.VectorSubcoreMesh(core_axis_name="c", subcore_axis_name="s")

    @functools.partial(
        pl.kernel,
        out_type=jax.ShapeDtypeStruct((NC, Npad, D), jnp.float32),
        mesh=mesh,
        compiler_params=pltpu.CompilerParams(use_tc_tiling_on_sc=False),
        scratch_types=[
            pltpu.VMEM((K, CHUNK), jnp.int32),
            pltpu.VMEM((CHUNK, D), jnp.float32),
            pltpu.VMEM((NZT, D), jnp.float32),
            pltpu.VMEM_SHARED((Npad, D), jnp.float32),
        ],
    )
    def kfn(dst_hbm, ones_hbm, zeros_hbm, out_hbm, dst_v, ones_v, zslab, acc):
        cid = lax.axis_index("c")
        sid = lax.axis_index("s")
        wid = cid * NS + sid
        pltpu.sync_copy(dst_hbm.at[wid], dst_v)
        pltpu.sync_copy(ones_hbm, ones_v)
        pltpu.sync_copy(zeros_hbm, zslab)
        pltpu.sync_copy(zslab, acc.at[pl.ds(sid * NZT, NZT)])
        plsc.subcore_barrier()

        def body(k, carry):
            pltpu.sync_copy(ones_v, acc.at[dst_v.at[k]], add=True)
            return carry

        lax.fori_loop(0, K, body, 0)
        plsc.subcore_barrier()
        pltpu.sync_copy(acc.at[pl.ds(sid * NZT, NZT)], zslab)
        pltpu.sync_copy(zslab, out_hbm.at[cid].at[pl.ds(sid * NZT, NZT)])

    return kfn


def _tc_scale_mm(x, W, d0, d1):
    N, D_out = x.shape[0], W.shape[1]

    def body(x_ref, w_ref, d0_ref, d1_ref, g_ref, dinv_ref):
        deg = d0_ref[...] + d1_ref[...]
        dinv = jnp.where(deg > 0, lax.rsqrt(deg), 0.0)
        h = jnp.dot(x_ref[...], w_ref[...], preferred_element_type=jnp.float32)
        g_ref[...] = h * dinv
        dinv_ref[...] = dinv

    return pl.pallas_call(
        body,
        out_shape=(jax.ShapeDtypeStruct((N, D_out), jnp.float32),
                   jax.ShapeDtypeStruct((N, 1), jnp.float32)),
    )(x, W, d0, d1)


def _tc_mid(p0, p1, dinv, b1, W2):
    N, D_out = p0.shape[0], W2.shape[1]

    def body(p0_ref, p1_ref, dinv_ref, b1_ref, w2_ref, g_ref):
        dinv = dinv_ref[...]
        h = jnp.maximum(dinv * (p0_ref[...] + p1_ref[...]) + b1_ref[...], 0.0)
        g_ref[...] = dinv * jnp.dot(h, w2_ref[...], preferred_element_type=jnp.float32)

    return pl.pallas_call(
        body,
        out_shape=jax.ShapeDtypeStruct((N, D_out), jnp.float32),
    )(p0, p1, dinv, b1, W2)


def _tc_final(p0, p1, dinv, b2):
    N, D_out = p0.shape

    def body(p0_ref, p1_ref, dinv_ref, b2_ref, o_ref):
        o_ref[...] = dinv_ref[...] * (p0_ref[...] + p1_ref[...]) + b2_ref[...]

    return pl.pallas_call(
        body,
        out_shape=jax.ShapeDtypeStruct((N, D_out), jnp.float32),
    )(p0, p1, dinv, b2)


def kernel(x, edge_index, W1, b1, W2, b2):
    N = x.shape[0]
    E = edge_index.shape[1]
    D_HID = W1.shape[1]
    D_OUT = W2.shape[1]
    Npad = (N // 128 + 1) * 128
    trash = Npad - N

    per_w = -(-E // NW)
    K = -(-per_w // CHUNK)
    K += K % 2
    Ep = NW * K * CHUNK
    pad = Ep - E
    it = jnp.arange(pad, dtype=jnp.int32)
    src_p = jnp.concatenate([edge_index[0], it % N]).reshape(NW, K, CHUNK)
    dst_p = jnp.concatenate([edge_index[1], N + (it % trash)]).reshape(NW, K, CHUNK)

    NZT = Npad // NS
    ones16 = jnp.ones((CHUNK, 16), jnp.float32)
    zeros16 = jnp.zeros((NZT, 16), jnp.float32)
    zeros_h = jnp.zeros((NZT, D_HID), jnp.float32)
    zeros_o = jnp.zeros((NZT, D_OUT), jnp.float32)

    pdeg = _deg_count(Npad, K)(dst_p, ones16, zeros16)
    g1, dinv = _tc_scale_mm(x, W1, pdeg[0, :N, 0:1], pdeg[1, :N, 0:1])
    p1 = _seg_sum(Npad, K, D_HID)(g1, src_p, dst_p, zeros_h)
    g2 = _tc_mid(p1[0, :N], p1[1, :N], dinv, b1.reshape(1, -1), W2)
    p2 = _seg_sum(Npad, K, D_OUT)(g2, src_p, dst_p, zeros_o)
    out = _tc_final(p2[0, :N], p2[1, :N], dinv, b2.reshape(1, -1))
    return (out, 0)


# speedup vs baseline: 36.9663x; 1.0718x over previous
import functools

import jax
import jax.numpy as jnp
from jax import lax
from jax.experimental import pallas as pl
from jax.experimental.pallas import tpu as pltpu
from jax.experimental.pallas import tpu_sc as plsc

NC = 2
NS = 16
NW = NC * NS
CHUNK = 128


def _seg_sum(Npad, K, D):
    NZT = Npad // NS
    mesh = plsc.VectorSubcoreMesh(core_axis_name="c", subcore_axis_name="s")

    @functools.partial(
        pl.kernel,
        out_type=jax.ShapeDtypeStruct((NC, Npad, D), jnp.float32),
        mesh=mesh,
        compiler_params=pltpu.CompilerParams(use_tc_tiling_on_sc=False),
        scratch_types=[
            pltpu.VMEM((K, CHUNK), jnp.int32),
            pltpu.VMEM((K, CHUNK), jnp.int32),
            pltpu.VMEM((4, CHUNK, D), jnp.float32),
            pltpu.VMEM((NZT, D), jnp.float32),
            pltpu.VMEM_SHARED((Npad, D), jnp.float32),
            [pltpu.SemaphoreType.DMA] * 4,
            [pltpu.SemaphoreType.DMA] * 4,
            pltpu.SemaphoreType.DMA,
            pltpu.SemaphoreType.DMA,
        ],
    )
    def kfn(g_hbm, src_hbm, dst_hbm, zeros_hbm, out_hbm,
            src_v, dst_v, ring, zslab, acc, gsems, ssems, semA, semB):
        cid = lax.axis_index("c")
        sid = lax.axis_index("s")
        wid = cid * NS + sid
        pltpu.async_copy(src_hbm.at[wid], src_v, semA)
        pltpu.async_copy(dst_hbm.at[wid], dst_v, semB)
        pltpu.sync_copy(zeros_hbm, zslab)
        pltpu.sync_copy(zslab, acc.at[pl.ds(sid * NZT, NZT)])
        pltpu.make_async_copy(src_hbm.at[wid], src_v, semA).wait()
        pltpu.make_async_copy(dst_hbm.at[wid], dst_v, semB).wait()
        plsc.subcore_barrier()

        def gather(k, b):
            pltpu.async_copy(g_hbm.at[src_v.at[k]], ring.at[b], gsems[b])

        def wait_gather(b):
            pltpu.make_async_copy(g_hbm.at[src_v.at[0]], ring.at[b],
                                  gsems[b]).wait()

        def scatter(k, b):
            pltpu.async_copy(ring.at[b], acc.at[dst_v.at[k]], ssems[b],
                             add=True)

        def wait_scatter(b):
            pltpu.make_async_copy(ring.at[b], acc.at[dst_v.at[0]],
                                  ssems[b]).wait()

        gather(0, 0)
        gather(1, 1)
        wait_gather(0); scatter(0, 0); gather(2, 2)
        wait_gather(1); scatter(1, 1); gather(3, 3)

        def body(q, carry):
            k0 = 4 * q + 2
            for i in range(4):
                b = (2 + i) % 4
                b2 = (b + 2) % 4
                wait_gather(b)
                scatter(k0 + i, b)
                wait_scatter(b2)
                gather(k0 + i + 2, b2)
            return carry

        lax.fori_loop(0, (K - 4) // 4, body, 0)

        wait_gather(2); scatter(K - 2, 2)
        wait_gather(3); scatter(K - 1, 3)
        for b in range(4):
            wait_scatter(b)
        plsc.subcore_barrier()
        pltpu.sync_copy(acc.at[pl.ds(sid * NZT, NZT)], zslab)
        pltpu.sync_copy(zslab, out_hbm.at[cid].at[pl.ds(sid * NZT, NZT)])

    return kfn


def _deg_count(Npad, K):
    D = 16
    NZT = Npad // NS
    mesh = plsc.VectorSubcoreMesh(core_axis_name="c", subcore_axis_name="s")

    @functools.partial(
        pl.kernel,
        out_type=jax.ShapeDtypeStruct((NC, Npad, D), jnp.float32),
        mesh=mesh,
        compiler_params=pltpu.CompilerParams(use_tc_tiling_on_sc=False),
        scratch_types=[
            pltpu.VMEM((K, CHUNK), jnp.int32),
            pltpu.VMEM((CHUNK, D), jnp.float32),
            pltpu.VMEM((NZT, D), jnp.float32),
            pltpu.VMEM_SHARED((Npad, D), jnp.float32),
            pltpu.SemaphoreType.DMA,
        ],
    )
    def kfn(dst_hbm, ones_hbm, zeros_hbm, out_hbm, dst_v, ones_v, zslab, acc,
            ssem):
        cid = lax.axis_index("c")
        sid = lax.axis_index("s")
        wid = cid * NS + sid
        pltpu.sync_copy(dst_hbm.at[wid], dst_v)
        pltpu.sync_copy(ones_hbm, ones_v)
        pltpu.sync_copy(zeros_hbm, zslab)
        pltpu.sync_copy(zslab, acc.at[pl.ds(sid * NZT, NZT)])
        plsc.subcore_barrier()

        def fire(k, carry):
            pltpu.async_copy(ones_v, acc.at[dst_v.at[k]], ssem, add=True)
            return carry

        lax.fori_loop(0, K, fire, 0)

        def drain(k, carry):
            pltpu.make_async_copy(ones_v, acc.at[dst_v.at[0]], ssem).wait()
            return carry

        lax.fori_loop(0, K, drain, 0)
        plsc.subcore_barrier()
        pltpu.sync_copy(acc.at[pl.ds(sid * NZT, NZT)], zslab)
        pltpu.sync_copy(zslab, out_hbm.at[cid].at[pl.ds(sid * NZT, NZT)])

    return kfn


def _tc_scale_mm(x, W, d0, d1):
    N, D_out = x.shape[0], W.shape[1]

    def body(x_ref, w_ref, d0_ref, d1_ref, g_ref, dinv_ref):
        deg = d0_ref[...] + d1_ref[...]
        dinv = jnp.where(deg > 0, lax.rsqrt(deg), 0.0)
        h = jnp.dot(x_ref[...], w_ref[...], preferred_element_type=jnp.float32)
        g_ref[...] = h * dinv
        dinv_ref[...] = dinv

    return pl.pallas_call(
        body,
        out_shape=(jax.ShapeDtypeStruct((N, D_out), jnp.float32),
                   jax.ShapeDtypeStruct((N, 1), jnp.float32)),
    )(x, W, d0, d1)


def _tc_mid(p0, p1, dinv, b1, W2):
    N, D_out = p0.shape[0], W2.shape[1]

    def body(p0_ref, p1_ref, dinv_ref, b1_ref, w2_ref, g_ref):
        dinv = dinv_ref[...]
        h = jnp.maximum(dinv * (p0_ref[...] + p1_ref[...]) + b1_ref[...], 0.0)
        g_ref[...] = dinv * jnp.dot(h, w2_ref[...], preferred_element_type=jnp.float32)

    return pl.pallas_call(
        body,
        out_shape=jax.ShapeDtypeStruct((N, D_out), jnp.float32),
    )(p0, p1, dinv, b1, W2)


def _tc_final(p0, p1, dinv, b2):
    N, D_out = p0.shape

    def body(p0_ref, p1_ref, dinv_ref, b2_ref, o_ref):
        o_ref[...] = dinv_ref[...] * (p0_ref[...] + p1_ref[...]) + b2_ref[...]

    return pl.pallas_call(
        body,
        out_shape=jax.ShapeDtypeStruct((N, D_out), jnp.float32),
    )(p0, p1, dinv, b2)


def kernel(x, edge_index, W1, b1, W2, b2):
    N = x.shape[0]
    E = edge_index.shape[1]
    D_HID = W1.shape[1]
    D_OUT = W2.shape[1]
    Npad = (N // 128 + 1) * 128
    trash = Npad - N

    per_w = -(-E // NW)
    K = -(-per_w // CHUNK)
    K += (-K) % 4
    Ep = NW * K * CHUNK
    pad = Ep - E
    it = jnp.arange(pad, dtype=jnp.int32)
    src_p = jnp.concatenate([edge_index[0], it % N]).reshape(NW, K, CHUNK)
    dst_p = jnp.concatenate([edge_index[1], N + (it % trash)]).reshape(NW, K, CHUNK)

    NZT = Npad // NS
    ones16 = jnp.ones((CHUNK, 16), jnp.float32)
    zeros16 = jnp.zeros((NZT, 16), jnp.float32)
    zeros_h = jnp.zeros((NZT, D_HID), jnp.float32)
    zeros_o = jnp.zeros((NZT, D_OUT), jnp.float32)

    pdeg = _deg_count(Npad, K)(dst_p, ones16, zeros16)
    g1, dinv = _tc_scale_mm(x, W1, pdeg[0, :N, 0:1], pdeg[1, :N, 0:1])
    p1 = _seg_sum(Npad, K, D_HID)(g1, src_p, dst_p, zeros_h)
    g2 = _tc_mid(p1[0, :N], p1[1, :N], dinv, b1.reshape(1, -1), W2)
    p2 = _seg_sum(Npad, K, D_OUT)(g2, src_p, dst_p, zeros_o)
    out = _tc_final(p2[0, :N], p2[1, :N], dinv, b2.reshape(1, -1))
    return (out, 0)


# speedup vs baseline: 40.3207x; 1.0907x over previous
import functools

import jax
import jax.numpy as jnp
from jax import lax
from jax.experimental import pallas as pl
from jax.experimental.pallas import tpu as pltpu
from jax.experimental.pallas import tpu_sc as plsc

NC = 2
NS = 16
NW = NC * NS


def _seg_sum(Npad, K, CHUNK, D):
    NZT = Npad // NS
    mesh = plsc.VectorSubcoreMesh(core_axis_name="c", subcore_axis_name="s")

    @functools.partial(
        pl.kernel,
        out_type=jax.ShapeDtypeStruct((NC, Npad, D), jnp.float32),
        mesh=mesh,
        compiler_params=pltpu.CompilerParams(use_tc_tiling_on_sc=False),
        scratch_types=[
            pltpu.VMEM((K, CHUNK), jnp.int32),
            pltpu.VMEM((K, CHUNK), jnp.int32),
            pltpu.VMEM((4, CHUNK, D), jnp.float32),
            pltpu.VMEM((NZT, D), jnp.float32),
            pltpu.VMEM_SHARED((Npad, D), jnp.float32),
            [pltpu.SemaphoreType.DMA] * 4,
            [pltpu.SemaphoreType.DMA] * 4,
            pltpu.SemaphoreType.DMA,
            pltpu.SemaphoreType.DMA,
        ],
    )
    def kfn(g_hbm, src_hbm, dst_hbm, zeros_hbm, out_hbm,
            src_v, dst_v, ring, zslab, acc, gsems, ssems, semA, semB):
        cid = lax.axis_index("c")
        sid = lax.axis_index("s")
        wid = cid * NS + sid
        pltpu.async_copy(src_hbm.at[wid], src_v, semA)
        pltpu.async_copy(dst_hbm.at[wid], dst_v, semB)
        pltpu.sync_copy(zeros_hbm, zslab)
        pltpu.sync_copy(zslab, acc.at[pl.ds(sid * NZT, NZT)])
        pltpu.make_async_copy(src_hbm.at[wid], src_v, semA).wait()
        pltpu.make_async_copy(dst_hbm.at[wid], dst_v, semB).wait()
        plsc.subcore_barrier()

        def gather(k, b):
            pltpu.async_copy(g_hbm.at[src_v.at[k]], ring.at[b], gsems[b])

        def wait_gather(b):
            pltpu.make_async_copy(g_hbm.at[src_v.at[0]], ring.at[b],
                                  gsems[b]).wait()

        def scatter(k, b):
            pltpu.async_copy(ring.at[b], acc.at[dst_v.at[k]], ssems[b],
                             add=True)

        def wait_scatter(b):
            pltpu.make_async_copy(ring.at[b], acc.at[dst_v.at[0]],
                                  ssems[b]).wait()

        gather(0, 0)
        gather(1, 1)
        wait_gather(0); scatter(0, 0); gather(2, 2)
        wait_gather(1); scatter(1, 1); gather(3, 3)

        def body(q, carry):
            k0 = 4 * q + 2
            for i in range(4):
                b = (2 + i) % 4
                b2 = (b + 2) % 4
                wait_gather(b)
                scatter(k0 + i, b)
                wait_scatter(b2)
                gather(k0 + i + 2, b2)
            return carry

        lax.fori_loop(0, (K - 4) // 4, body, 0)

        wait_gather(2); scatter(K - 2, 2)
        wait_gather(3); scatter(K - 1, 3)
        for b in range(4):
            wait_scatter(b)
        plsc.subcore_barrier()
        pltpu.sync_copy(acc.at[pl.ds(sid * NZT, NZT)], zslab)
        pltpu.sync_copy(zslab, out_hbm.at[cid].at[pl.ds(sid * NZT, NZT)])

    return kfn


def _deg_count(Npad, K, CHUNK):
    D = 16
    NZT = Npad // NS
    mesh = plsc.VectorSubcoreMesh(core_axis_name="c", subcore_axis_name="s")

    @functools.partial(
        pl.kernel,
        out_type=jax.ShapeDtypeStruct((NC, Npad, D), jnp.float32),
        mesh=mesh,
        compiler_params=pltpu.CompilerParams(use_tc_tiling_on_sc=False),
        scratch_types=[
            pltpu.VMEM((K, CHUNK), jnp.int32),
            pltpu.VMEM((CHUNK, D), jnp.float32),
            pltpu.VMEM((NZT, D), jnp.float32),
            pltpu.VMEM_SHARED((Npad, D), jnp.float32),
            pltpu.SemaphoreType.DMA,
        ],
    )
    def kfn(dst_hbm, ones_hbm, zeros_hbm, out_hbm, dst_v, ones_v, zslab, acc,
            ssem):
        cid = lax.axis_index("c")
        sid = lax.axis_index("s")
        wid = cid * NS + sid
        pltpu.sync_copy(dst_hbm.at[wid], dst_v)
        pltpu.sync_copy(ones_hbm, ones_v)
        pltpu.sync_copy(zeros_hbm, zslab)
        pltpu.sync_copy(zslab, acc.at[pl.ds(sid * NZT, NZT)])
        plsc.subcore_barrier()

        def fire(k, carry):
            pltpu.async_copy(ones_v, acc.at[dst_v.at[k]], ssem, add=True)
            return carry

        lax.fori_loop(0, K, fire, 0)

        def drain(k, carry):
            pltpu.make_async_copy(ones_v, acc.at[dst_v.at[0]], ssem).wait()
            return carry

        lax.fori_loop(0, K, drain, 0)
        plsc.subcore_barrier()
        pltpu.sync_copy(acc.at[pl.ds(sid * NZT, NZT)], zslab)
        pltpu.sync_copy(zslab, out_hbm.at[cid].at[pl.ds(sid * NZT, NZT)])

    return kfn


def _tc_scale_mm(x, W, pdeg):
    N, D_out = x.shape[0], W.shape[1]

    def body(x_ref, w_ref, pd_ref, g_ref, dinv_ref):
        deg = pd_ref[0, :N, 0:1] + pd_ref[1, :N, 0:1]
        dinv = jnp.where(deg > 0, lax.rsqrt(deg), 0.0)
        h = jnp.dot(x_ref[...], w_ref[...], preferred_element_type=jnp.float32)
        g_ref[...] = h * dinv
        dinv_ref[...] = dinv

    return pl.pallas_call(
        body,
        out_shape=(jax.ShapeDtypeStruct((N, D_out), jnp.float32),
                   jax.ShapeDtypeStruct((N, 1), jnp.float32)),
    )(x, W, pdeg)


def _tc_mid(p1, dinv, b1, W2):
    N, D_out = dinv.shape[0], W2.shape[1]

    def body(p_ref, dinv_ref, b1_ref, w2_ref, g_ref):
        dinv = dinv_ref[...]
        s = p_ref[0, :N, :] + p_ref[1, :N, :]
        h = jnp.maximum(dinv * s + b1_ref[...], 0.0)
        g_ref[...] = dinv * jnp.dot(h, w2_ref[...],
                                    preferred_element_type=jnp.float32)

    return pl.pallas_call(
        body,
        out_shape=jax.ShapeDtypeStruct((N, D_out), jnp.float32),
    )(p1, dinv, b1, W2)


def _tc_final(p2, dinv, b2):
    N = dinv.shape[0]
    D_out = p2.shape[2]

    def body(p_ref, dinv_ref, b2_ref, o_ref):
        s = p_ref[0, :N, :] + p_ref[1, :N, :]
        o_ref[...] = dinv_ref[...] * s + b2_ref[...]

    return pl.pallas_call(
        body,
        out_shape=jax.ShapeDtypeStruct((N, D_out), jnp.float32),
    )(p2, dinv, b2)


def kernel(x, edge_index, W1, b1, W2, b2):
    N = x.shape[0]
    E = edge_index.shape[1]
    D_HID = W1.shape[1]
    D_OUT = W2.shape[1]
    Npad = (N // 128 + 1) * 128

    per_w = -(-E // NW)
    CHUNK = 0
    for c in range(128, 63, -1):
        if per_w * NW == E and per_w % c == 0 and (per_w // c) % 4 == 0:
            CHUNK = c
            break
    if CHUNK:
        K = per_w // CHUNK
        src_p = edge_index[0].reshape(NW, K, CHUNK)
        dst_p = edge_index[1].reshape(NW, K, CHUNK)
    else:
        CHUNK = 128
        K = -(-per_w // CHUNK)
        K += (-K) % 4
        pad = NW * K * CHUNK - E
        it = jnp.arange(pad, dtype=jnp.int32)
        src_p = jnp.concatenate([edge_index[0], it % N]).reshape(NW, K, CHUNK)
        dst_p = jnp.concatenate([edge_index[1], N + (it % (Npad - N))]
                                ).reshape(NW, K, CHUNK)

    NZT = Npad // NS
    ones16 = jnp.ones((CHUNK, 16), jnp.float32)
    zeros16 = jnp.zeros((NZT, 16), jnp.float32)
    zeros_h = jnp.zeros((NZT, D_HID), jnp.float32)
    zeros_o = jnp.zeros((NZT, D_OUT), jnp.float32)

    pdeg = _deg_count(Npad, K, CHUNK)(dst_p, ones16, zeros16)
    g1, dinv = _tc_scale_mm(x, W1, pdeg)
    p1 = _seg_sum(Npad, K, CHUNK, D_HID)(g1, src_p, dst_p, zeros_h)
    g2 = _tc_mid(p1, dinv, b1.reshape(1, -1), W2)
    p2 = _seg_sum(Npad, K, CHUNK, D_OUT)(g2, src_p, dst_p, zeros_o)
    out = _tc_final(p2, dinv, b2.reshape(1, -1))
    return (out, 0)
